# baseline (speedup 1.0000x reference)
import jax
import jax.numpy as jnp
from jax.experimental import pallas as pl
from jax.experimental.pallas import tpu as pltpu

_N = 16


def _body(x_ref, y_ref, X_ref, y_hbm, o_hbm, rowbuf, copy_sem, row_sem):
    n = rowbuf.shape[0]
    big = pltpu.make_async_copy(y_hbm, o_hbm, copy_sem)
    big.start()

    row_dmas = []
    for i in range(n):
        r = y_ref[i]
        d = pltpu.make_async_copy(
            y_hbm.at[i, pl.ds(r, 1), :], rowbuf.at[i], row_sem
        )
        d.start()
        row_dmas.append(d)
    w = rowbuf.shape[-1]
    cols = jax.lax.broadcasted_iota(jnp.int32, (1, w), 1)
    for i in range(n):
        row_dmas[i].wait()
        c = x_ref[i]
        rowbuf[i] = rowbuf[i] + jnp.where(cols == c, X_ref[0], 0.0)

    big.wait()
    wb_dmas = []
    for i in range(n):
        r = y_ref[i]
        d = pltpu.make_async_copy(
            rowbuf.at[i], o_hbm.at[i, pl.ds(r, 1), :], row_sem
        )
        d.start()
        wb_dmas.append(d)
    for d in wb_dmas:
        d.wait()


def kernel(Y, X, x, y):
    n, h, w = Y.shape
    return pl.pallas_call(
        _body,
        in_specs=[
            pl.BlockSpec(memory_space=pltpu.SMEM),
            pl.BlockSpec(memory_space=pltpu.SMEM),
            pl.BlockSpec(memory_space=pltpu.SMEM),
            pl.BlockSpec(memory_space=pl.ANY),
        ],
        out_specs=pl.BlockSpec(memory_space=pl.ANY),
        out_shape=jax.ShapeDtypeStruct(Y.shape, Y.dtype),
        scratch_shapes=[
            pltpu.VMEM((n, 1, w), jnp.float32),
            pltpu.SemaphoreType.DMA,
            pltpu.SemaphoreType.DMA,
        ],
    )(x, y, X, Y)


# speedup vs baseline: 48.8979x; 48.8979x over previous
import jax
import jax.numpy as jnp
from jax.experimental import pallas as pl
from jax.experimental.pallas import tpu as pltpu

_BH = 1024


def _body(x_ref, y_ref, X_ref, y_blk, o_blk):
    i = pl.program_id(0)
    j = pl.program_id(1)
    o_blk[...] = y_blk[...]
    r_loc = y_ref[i] - j * _BH
    c = x_ref[i]

    @pl.when((r_loc >= 0) & (r_loc < _BH))
    def _inject():
        row = y_blk[0, pl.ds(r_loc, 1), :]
        w = row.shape[-1]
        colmask = jax.lax.broadcasted_iota(jnp.int32, (1, w), 1) == c
        o_blk[0, pl.ds(r_loc, 1), :] = row + jnp.where(colmask, X_ref[0], 0.0)


def kernel(Y, X, x, y):
    n, h, w = Y.shape
    grid = (n, h // _BH)
    return pl.pallas_call(
        _body,
        grid=grid,
        in_specs=[
            pl.BlockSpec(memory_space=pltpu.SMEM),
            pl.BlockSpec(memory_space=pltpu.SMEM),
            pl.BlockSpec(memory_space=pltpu.SMEM),
            pl.BlockSpec((1, _BH, w), lambda i, j: (i, j, 0)),
        ],
        out_specs=pl.BlockSpec((1, _BH, w), lambda i, j: (i, j, 0)),
        out_shape=jax.ShapeDtypeStruct(Y.shape, Y.dtype),
        compiler_params=pltpu.CompilerParams(
            dimension_semantics=("parallel", "parallel"),
        ),
    )(x, y, X, Y)
